# baseline (speedup 1.0000x reference)
import functools

import jax
import jax.numpy as jnp
from jax import lax
from jax.experimental import pallas as pl
from jax.experimental.pallas import tpu as pltpu
from jax.experimental.pallas import tpu_sc as plsc

NC = 2
NS = 16
NW = NC * NS
G = 128
K = 8


def _emb_call(n_groups, g_per_w, n_outer, V, D, N):
    mesh = plsc.VectorSubcoreMesh(core_axis_name="c", subcore_axis_name="s")

    @functools.partial(
        pl.kernel,
        mesh=mesh,
        out_type=jax.ShapeDtypeStruct((N, D), jnp.float32),
        scratch_types=[
            pltpu.VMEM((K, G), jnp.int32),
            pltpu.VMEM((K * G, D), jnp.float32),
            pltpu.SemaphoreType.DMA,
        ],
        compiler_params=pltpu.CompilerParams(use_tc_tiling_on_sc=False),
    )
    def emb(idx_hbm, table_hbm, out_hbm, idx_v, rows_v, sem):
        wid = lax.axis_index("s") * NC + lax.axis_index("c")
        g0 = wid * g_per_w

        def step(i, carry):
            row = g0 + i * K
            pltpu.sync_copy(idx_hbm.at[pl.ds(row, K), :], idx_v)
            handles = []
            for j in range(K):
                handles.append(
                    pltpu.async_copy(
                        table_hbm.at[idx_v.at[j]],
                        rows_v.at[pl.ds(j * G, G)],
                        sem,
                    )
                )
            for h in handles:
                h.wait()
            pltpu.sync_copy(rows_v, out_hbm.at[pl.ds(row * G, K * G)])
            return carry

        lax.fori_loop(0, n_outer, step, 0)

    return emb


def kernel(x, table):
    B, H = x.shape
    V, D = table.shape
    N = B * H
    n_groups = N // G
    g_per_w = n_groups // NW
    n_outer = g_per_w // K
    idx2d = x.reshape(n_groups, G).astype(jnp.int32)
    out = _emb_call(n_groups, g_per_w, n_outer, V, D, N)(idx2d, table)
    return out.reshape(B, H, D)


# speedup vs baseline: 1.0092x; 1.0092x over previous
import functools

import jax
import jax.numpy as jnp
from jax import lax
from jax.experimental import pallas as pl
from jax.experimental.pallas import tpu as pltpu
from jax.experimental.pallas import tpu_sc as plsc

NC = 2
NS = 16
NW = NC * NS
G = 128
K = 5
NBUF = 2


def _emb_call(g_per_w, n_outer, D, N):
    mesh = plsc.VectorSubcoreMesh(core_axis_name="c", subcore_axis_name="s")

    @functools.partial(
        pl.kernel,
        mesh=mesh,
        out_type=jax.ShapeDtypeStruct((N, D), jnp.float32),
        scratch_types=[
            [pltpu.VMEM((K * G,), jnp.int32) for _ in range(NBUF)],
            [pltpu.VMEM((K * G, D), jnp.float32) for _ in range(NBUF)],
            [pltpu.SemaphoreType.DMA for _ in range(NBUF)],
            [pltpu.SemaphoreType.DMA for _ in range(NBUF)],
        ],
        compiler_params=pltpu.CompilerParams(use_tc_tiling_on_sc=False),
    )
    def emb(idx_hbm, table_hbm, out_hbm, idx_v, rows_v, sem_g, sem_o):
        wid = lax.axis_index("s") * NC + lax.axis_index("c")
        g0 = wid * g_per_w

        def out_copy(i, b):
            return pltpu.make_async_copy(
                rows_v[b],
                out_hbm.at[pl.ds((g0 + i * K) * G, K * G)],
                sem_o[b],
            )

        def step(i, b):
            row = g0 + i * K
            pltpu.sync_copy(idx_hbm.at[pl.ds(row * G, K * G)], idx_v[b])

            @pl.when(i >= NBUF)
            def _():
                out_copy(i - NBUF, b).wait()

            handles = [
                pltpu.async_copy(
                    table_hbm.at[idx_v[b].at[pl.ds(j * G, G)]],
                    rows_v[b].at[pl.ds(j * G, G)],
                    sem_g[b],
                )
                for j in range(K)
            ]
            for h in handles:
                h.wait()
            out_copy(i, b).start()

        def outer(o, carry):
            for b in range(NBUF):
                step(o * NBUF + b, b)
            return carry

        lax.fori_loop(0, n_outer // NBUF, outer, 0)
        for b in range(NBUF):
            out_copy(n_outer - NBUF + b, b).wait()

    return emb


def kernel(x, table):
    B, H = x.shape
    V, D = table.shape
    N = B * H
    n_groups = N // G
    g_per_w = n_groups // NW
    n_outer = g_per_w // K
    idx_flat = x.reshape(N).astype(jnp.int32)
    out = _emb_call(g_per_w, n_outer, D, N)(idx_flat, table)
    return out.reshape(B, H, D)


# speedup vs baseline: 1.0188x; 1.0095x over previous
import functools

import jax
import jax.numpy as jnp
from jax import lax
from jax.experimental import pallas as pl
from jax.experimental.pallas import tpu as pltpu
from jax.experimental.pallas import tpu_sc as plsc

NC = 2
NS = 16
NW = NC * NS
C = 800
NBUF = 2


def _emb_call(c_per_w, n_steps, D, N):
    mesh = plsc.VectorSubcoreMesh(core_axis_name="c", subcore_axis_name="s")

    @functools.partial(
        pl.kernel,
        mesh=mesh,
        out_type=jax.ShapeDtypeStruct((N, D), jnp.float32),
        scratch_types=[
            [pltpu.VMEM((C,), jnp.int32) for _ in range(NBUF)],
            [pltpu.VMEM((C, D), jnp.float32) for _ in range(NBUF)],
            [pltpu.SemaphoreType.DMA for _ in range(NBUF)],
            [pltpu.SemaphoreType.DMA for _ in range(NBUF)],
        ],
        compiler_params=pltpu.CompilerParams(use_tc_tiling_on_sc=False),
    )
    def emb(idx_hbm, table_hbm, out_hbm, idx_v, rows_v, sem_g, sem_o):
        wid = lax.axis_index("s") * NC + lax.axis_index("c")
        r0 = wid * c_per_w

        def gather_copy(i, b):
            return pltpu.make_async_copy(
                table_hbm.at[idx_v[b]], rows_v[b], sem_g[b])

        def out_copy(i, b):
            return pltpu.make_async_copy(
                rows_v[b], out_hbm.at[pl.ds(r0 + i * C, C)], sem_o[b])

        def fire(i, b, wait_out):
            pltpu.sync_copy(idx_hbm.at[pl.ds(r0 + i * C, C)], idx_v[b])
            if wait_out:
                out_copy(i - NBUF, b).wait()
            gather_copy(i, b).start()

        def retire(i, b):
            gather_copy(i, b).wait()
            out_copy(i, b).start()

        fire(0, 0, False)
        fire(1, 1, False)
        retire(0, 0)

        def steady(o, carry):
            for k in range(NBUF):
                i = o * NBUF + k
                fire(i, k, True)
                retire(i - 1, (k - 1) % NBUF)
            return carry

        lax.fori_loop(1, n_steps // NBUF, steady, 0)
        retire(n_steps - 1, (n_steps - 1) % NBUF)
        for i in range(n_steps - NBUF, n_steps):
            out_copy(i, i % NBUF).wait()

    return emb


def kernel(x, table):
    B, H = x.shape
    V, D = table.shape
    N = B * H
    c_per_w = N // NW
    n_steps = c_per_w // C
    idx_flat = x.reshape(N).astype(jnp.int32)
    out = _emb_call(c_per_w, n_steps, D, N)(idx_flat, table)
    return out.reshape(B, H, D)
